# baseline (speedup 1.0000x reference)
import functools

import jax
import jax.numpy as jnp
from jax import lax
from jax.experimental import pallas as pl
from jax.experimental.pallas import tpu as pltpu
from jax.experimental.pallas import tpu_sc as plsc

D = 1024
B = 16384
NC = 2
NS = 16
NW = NC * NS
CHUNK = 8
PER_W = B // NW
NCHUNK = PER_W // CHUNK
NBUF = 12
G = 4
NPAIR = NBUF // G


def _make_emb():
    mesh = plsc.VectorSubcoreMesh(core_axis_name="c", subcore_axis_name="s")

    @functools.partial(
        pl.kernel,
        mesh=mesh,
        out_type=jax.ShapeDtypeStruct((NW * NCHUNK, CHUNK, D), jnp.float32),
        scratch_types=(
            [pltpu.VMEM((NCHUNK, CHUNK), jnp.int32),
             pltpu.VMEM((NBUF, CHUNK, D), jnp.float32)]
            + [pltpu.SemaphoreType.DMA] * (NBUF + NPAIR)
        ),
    )
    def emb(ids_hbm, table_hbm, out_hbm, idx_v, rows_all, *sems):
        gsem = sems[:NBUF]
        psem = sems[NBUF:]
        wid = lax.axis_index("s") * NC + lax.axis_index("c")
        pltpu.sync_copy(ids_hbm.at[pl.ds(wid * NCHUNK, NCHUNK)], idx_v)

        g = [None] * NBUF
        ps = [None] * NPAIR
        out_base = wid * NCHUNK

        for j in range(NBUF - G):
            g[j] = pltpu.async_copy(table_hbm.at[idx_v.at[j]],
                                    rows_all.at[j], gsem[j])
        for j in range(NCHUNK):
            b = j % NBUF
            jn = j + NBUF - G
            if jn < NCHUNK:
                nb = jn % NBUF
                q = nb // G
                if ps[q] is not None:
                    ps[q].wait()
                    ps[q] = None
                g[nb] = pltpu.async_copy(
                    table_hbm.at[idx_v.at[jn]], rows_all.at[nb], gsem[nb])
            g[b].wait()
            if j % G == G - 1:
                pb = b // G
                ps[pb] = pltpu.async_copy(
                    rows_all.at[pl.ds(G * pb, G)],
                    out_hbm.at[pl.ds(out_base + j - G + 1, G)], psem[pb])
        for pb in range(NPAIR):
            if ps[pb] is not None:
                ps[pb].wait()

    return emb


_emb = _make_emb()


@jax.jit
def kernel(input_ids, table):
    ids = input_ids.reshape(NW * NCHUNK, CHUNK).astype(jnp.int32)
    out = _emb(ids, table)
    return out.reshape(input_ids.shape + (D,))


# speedup vs baseline: 1.0298x; 1.0298x over previous
import functools

import jax
import jax.numpy as jnp
from jax import lax
from jax.experimental import pallas as pl
from jax.experimental.pallas import tpu as pltpu
from jax.experimental.pallas import tpu_sc as plsc

D = 1024
B = 16384
NC = 2
NS = 16
NW = NC * NS
CHUNK = 16
PER_W = B // NW
NCHUNK = PER_W // CHUNK
NBUF = 6
G = 2
NPAIR = NBUF // G


def _make_emb():
    mesh = plsc.VectorSubcoreMesh(core_axis_name="c", subcore_axis_name="s")

    @functools.partial(
        pl.kernel,
        mesh=mesh,
        out_type=jax.ShapeDtypeStruct((NW * NCHUNK, CHUNK, D), jnp.float32),
        scratch_types=(
            [pltpu.VMEM((NCHUNK, CHUNK), jnp.int32),
             pltpu.VMEM((NBUF, CHUNK, D), jnp.float32)]
            + [pltpu.SemaphoreType.DMA] * (NBUF + NPAIR)
        ),
    )
    def emb(ids_hbm, table_hbm, out_hbm, idx_v, rows_all, *sems):
        gsem = sems[:NBUF]
        psem = sems[NBUF:]
        wid = lax.axis_index("s") * NC + lax.axis_index("c")
        pltpu.sync_copy(ids_hbm.at[pl.ds(wid * NCHUNK, NCHUNK)], idx_v)

        g = [None] * NBUF
        ps = [None] * NPAIR
        out_base = wid * NCHUNK

        for j in range(NBUF - G):
            g[j] = pltpu.async_copy(table_hbm.at[idx_v.at[j]],
                                    rows_all.at[j], gsem[j])
        for j in range(NCHUNK):
            b = j % NBUF
            jn = j + NBUF - G
            if jn < NCHUNK:
                nb = jn % NBUF
                q = nb // G
                if ps[q] is not None:
                    ps[q].wait()
                    ps[q] = None
                g[nb] = pltpu.async_copy(
                    table_hbm.at[idx_v.at[jn]], rows_all.at[nb], gsem[nb])
            g[b].wait()
            if j % G == G - 1:
                pb = b // G
                ps[pb] = pltpu.async_copy(
                    rows_all.at[pl.ds(G * pb, G)],
                    out_hbm.at[pl.ds(out_base + j - G + 1, G)], psem[pb])
        for pb in range(NPAIR):
            if ps[pb] is not None:
                ps[pb].wait()

    return emb


_emb = _make_emb()


@jax.jit
def kernel(input_ids, table):
    ids = input_ids.reshape(NW * NCHUNK, CHUNK).astype(jnp.int32)
    out = _emb(ids, table)
    return out.reshape(input_ids.shape + (D,))
